# Initial kernel scaffold; baseline (speedup 1.0000x reference)
import jax
import jax.numpy as jnp
from jax.experimental import pallas as pl


def kernel(x_in, table):
    raise NotImplementedError("write your pallas kernel here")


# speedup vs baseline: 4.4800x; 4.4800x over previous
import functools

import numpy as np
import jax
import jax.numpy as jnp
from jax import lax
from jax.experimental import pallas as pl
from jax.experimental.pallas import tpu as pltpu
from jax.experimental.pallas import tpu_sc as plsc

_EMBED_DIM = 16
_NW = 32
_SUB = 100
_NSUB = 16
_CHUNK = _SUB * _NSUB


def _pos_encoding(max_pos, embed_dim):
    pos = np.arange(max_pos)[:, np.newaxis].astype(np.float32)
    i = np.arange(embed_dim)[np.newaxis, :].astype(np.float32)
    angle_rates = 1.0 / np.power(10000, 2 * (i // 2) / np.float32(embed_dim))
    angle_rads = pos * angle_rates
    angle_rads[:, 0::2] = np.sin(angle_rads[:, 0::2])
    angle_rads[:, 1::2] = np.cos(angle_rads[:, 1::2])
    return angle_rads


def kernel(x_in, table):
    B, S = x_in.shape
    D = table.shape[1]
    total = B * S
    n_chunks = total // _CHUNK
    per_w = n_chunks // _NW

    idx3 = x_in.reshape(n_chunks, _NSUB, _SUB).astype(jnp.int32)
    tmpl_np = np.tile(_pos_encoding(S, D), (_CHUNK // S, 1))
    tmpl = jnp.asarray(tmpl_np.reshape(_NSUB, _SUB, D), dtype=jnp.float32)

    mesh = plsc.VectorSubcoreMesh(core_axis_name="c", subcore_axis_name="s")

    @functools.partial(
        pl.kernel,
        mesh=mesh,
        compiler_params=pltpu.CompilerParams(use_tc_tiling_on_sc=False),
        out_type=jax.ShapeDtypeStruct((n_chunks, _NSUB, _SUB, D), jnp.float32),
        scratch_types=[
            pltpu.VMEM((_NSUB, _SUB), jnp.int32),
            pltpu.VMEM((_NSUB, _SUB, D), jnp.float32),
            pltpu.VMEM_SHARED((_NSUB, _SUB, D), jnp.float32),
            pltpu.SemaphoreType.DMA,
        ],
    )
    def k(idx_hbm, table_hbm, tmpl_hbm, out_hbm, idx_v, buf_v, tmpl_s, sem):
        sid = lax.axis_index("s")
        wid = sid * 2 + lax.axis_index("c")

        @pl.when(sid == 0)
        def _():
            pltpu.sync_copy(tmpl_hbm, tmpl_s)

        plsc.subcore_barrier()

        def body(c, _):
            chunk = wid * per_w + c
            pltpu.sync_copy(idx_hbm.at[chunk], idx_v)
            pltpu.sync_copy(tmpl_s, buf_v)
            descs = [
                pltpu.async_copy(
                    table_hbm.at[idx_v.at[j]], buf_v.at[j], sem, add=True
                )
                for j in range(_NSUB)
            ]
            for d in descs:
                d.wait()
            pltpu.sync_copy(buf_v, out_hbm.at[chunk])
            return 0

        lax.fori_loop(0, per_w, body, 0)

    out = k(idx3, table, tmpl)
    return out.reshape(B, S, D)


# speedup vs baseline: 4.7347x; 1.0568x over previous
import functools

import numpy as np
import jax
import jax.numpy as jnp
from jax import lax
from jax.experimental import pallas as pl
from jax.experimental.pallas import tpu as pltpu
from jax.experimental.pallas import tpu_sc as plsc

_NW = 32
_SUB = 100
_NSUB = 16
_CHUNK = _SUB * _NSUB
_NBUF = 4


def _pos_encoding(max_pos, embed_dim):
    pos = np.arange(max_pos)[:, np.newaxis].astype(np.float32)
    i = np.arange(embed_dim)[np.newaxis, :].astype(np.float32)
    angle_rates = 1.0 / np.power(10000, 2 * (i // 2) / np.float32(embed_dim))
    angle_rads = pos * angle_rates
    angle_rads[:, 0::2] = np.sin(angle_rads[:, 0::2])
    angle_rads[:, 1::2] = np.cos(angle_rads[:, 1::2])
    return angle_rads


def kernel(x_in, table):
    B, S = x_in.shape
    D = table.shape[1]
    total = B * S
    n_chunks = total // _CHUNK
    per_w = n_chunks // _NW

    idx3 = x_in.reshape(n_chunks, _NSUB, _SUB).astype(jnp.int32)
    tmpl_np = np.tile(_pos_encoding(S, D), (_CHUNK // S, 1))
    tmpl = jnp.asarray(tmpl_np.reshape(_NSUB, _SUB, D), dtype=jnp.float32)

    mesh = plsc.VectorSubcoreMesh(core_axis_name="c", subcore_axis_name="s")

    @functools.partial(
        pl.kernel,
        mesh=mesh,
        compiler_params=pltpu.CompilerParams(use_tc_tiling_on_sc=False),
        out_type=jax.ShapeDtypeStruct((n_chunks, _NSUB, _SUB, D), jnp.float32),
        scratch_types=[
            [pltpu.VMEM((_NSUB, _SUB), jnp.int32) for _ in range(_NBUF)],
            [pltpu.VMEM((_NSUB, _SUB, D), jnp.float32) for _ in range(_NBUF)],
            pltpu.VMEM_SHARED((_NSUB, _SUB, D), jnp.float32),
            [pltpu.SemaphoreType.DMA for _ in range(_NBUF)],
            [pltpu.SemaphoreType.DMA for _ in range(_NBUF)],
            [pltpu.SemaphoreType.DMA for _ in range(_NBUF)],
            [pltpu.SemaphoreType.DMA for _ in range(_NBUF)],
        ],
    )
    def k(idx_hbm, table_hbm, tmpl_hbm, out_hbm,
          idx_v, buf_v, tmpl_s, sem_i, sem_t, sem_g, sem_w):
        sid = lax.axis_index("s")
        wid = sid * 2 + lax.axis_index("c")
        base = wid * per_w

        @pl.when(sid == 0)
        def _():
            pltpu.sync_copy(tmpl_hbm, tmpl_s)

        plsc.subcore_barrier()

        def start_stage(c, p):
            pltpu.async_copy(idx_hbm.at[base + c], idx_v[p], sem_i[p])
            pltpu.async_copy(tmpl_s, buf_v[p], sem_t[p])

        def fire_gathers(p):
            pltpu.make_async_copy(idx_hbm.at[base], idx_v[p], sem_i[p]).wait()
            pltpu.make_async_copy(tmpl_s, buf_v[p], sem_t[p]).wait()
            return [
                pltpu.async_copy(
                    table_hbm.at[idx_v[p].at[j]], buf_v[p].at[j],
                    sem_g[p], add=True,
                )
                for j in range(_NSUB)
            ]

        def finish_stage(c, p, descs):
            for d in descs:
                d.wait()
            pltpu.async_copy(buf_v[p], out_hbm.at[base + c], sem_w[p])

        def wait_wb(p):
            pltpu.make_async_copy(buf_v[p], out_hbm.at[base], sem_w[p]).wait()

        for p in range(_NBUF):
            start_stage(p, p)

        n_rounds = per_w // _NBUF

        def body(g, _):
            c0 = g * _NBUF
            descs = [fire_gathers(p) for p in range(_NBUF)]
            for p in range(_NBUF):
                finish_stage(c0 + p, p, descs[p])

                @pl.when(g < n_rounds - 1)
                def _(p=p):
                    wait_wb(p)
                    start_stage(c0 + p + _NBUF, p)

            return 0

        lax.fori_loop(0, n_rounds, body, 0)
        for p in range(_NBUF):
            wait_wb(p)

    out = k(idx3, table, tmpl)
    return out.reshape(B, S, D)
